# baseline (speedup 1.0000x reference)
import functools

import jax
import jax.numpy as jnp
from jax import lax
from jax.experimental import pallas as pl
from jax.experimental.pallas import tpu as pltpu
from jax.experimental.pallas import tpu_sc as plsc

N = 8192
K = 20
NUM_GRAPHS = 8
F32 = jnp.float32
HIGH = lax.Precision.HIGHEST
MASKV = 1e37
NEDGE = N * K


def _dot(a, b):
    return lax.dot(a, b, precision=lax.Precision.DEFAULT,
                   preferred_element_type=F32)


def _knn_body(x_ref, xt_ref, br_ref, bc_ref, idx_ref, dist_s, m_s, j_s,
              *, BR, TW):
    xb = x_ref[...]
    sqr = jnp.sum(xb * xb, axis=1, keepdims=True)
    brow = br_ref[...]
    bfirst = br_ref[0, 0]
    blast = br_ref[BR - 1, 0]
    nt = N // TW
    c_lo = jnp.zeros((), jnp.int32)
    c_hi = jnp.zeros((), jnp.int32)
    for t in range(nt):
        bc_t = bc_ref[t]
        c_lo += jnp.sum((bc_t < bfirst).astype(jnp.int32))
        c_hi += jnp.sum((bc_t <= blast).astype(jnp.int32))
    t_lo = c_lo // TW
    t_hi = (c_hi + TW - 1) // TW

    for t in range(nt):
        @pl.when((t >= t_lo) & (t < t_hi))
        def _(t=t):
            xt = xt_ref[t]
            sqc = jnp.sum(xt * xt, axis=0, keepdims=True)
            d = sqr + sqc - 2.0 * _dot(xb, xt)
            dist_s[t] = jnp.where(brow != bc_ref[t], MASKV, d)

    m = jnp.full((BR, 1), -jnp.inf, F32)
    j = jnp.full((BR, 1), -1, jnp.int32)
    cols = []
    for _ in range(K):
        m_s[...] = jnp.full((BR, 1), jnp.inf, F32)
        j_s[...] = jnp.full((BR, 1), N, jnp.int32)
        for t in range(nt):
            @pl.when((t >= t_lo) & (t < t_hi))
            def _(t=t, m=m, j=j):
                v = dist_s[t]
                cid = lax.broadcasted_iota(jnp.int32, (BR, TW), 1) + t * TW
                elig = (v > m) | ((v == m) & (cid > j))
                vm = jnp.where(elig, v, jnp.inf)
                tmin = jnp.min(vm, axis=1, keepdims=True)
                targ = jnp.min(jnp.where(vm == tmin, cid, jnp.int32(N)),
                               axis=1, keepdims=True)
                mc = m_s[...]
                jc = j_s[...]
                j_s[...] = jnp.where(tmin < mc, targ,
                                     jnp.where(tmin == mc,
                                               jnp.minimum(jc, targ), jc))
                m_s[...] = jnp.minimum(mc, tmin)
        m = m_s[...]
        j = j_s[...]
        cols.append(jnp.minimum(j, N - 1))
    idx_ref[...] = jnp.concatenate(cols, axis=1)


def _knn(x, xt3, br, bc3):
    BR = 128
    TW = 512
    nt = N // TW
    f = x.shape[1]
    return pl.pallas_call(
        functools.partial(_knn_body, BR=BR, TW=TW),
        grid=(N // BR,),
        in_specs=[
            pl.BlockSpec((BR, f), lambda i: (i, 0)),
            pl.BlockSpec((nt, f, TW), lambda i: (0, 0, 0)),
            pl.BlockSpec((BR, 1), lambda i: (i, 0)),
            pl.BlockSpec((nt, 1, TW), lambda i: (0, 0, 0)),
        ],
        out_specs=pl.BlockSpec((BR, K), lambda i: (i, 0)),
        out_shape=jax.ShapeDtypeStruct((N, K), jnp.int32),
        scratch_shapes=[pltpu.VMEM((nt, BR, TW), F32),
                        pltpu.VMEM((BR, 1), F32),
                        pltpu.VMEM((BR, 1), jnp.int32)],
    )(x, xt3, br, bc3)


def _sc_gather(V1, idxf):
    mesh = plsc.VectorSubcoreMesh(core_axis_name="c", subcore_axis_name="s")
    CH = 128
    npts = N // 32

    @functools.partial(
        pl.kernel,
        mesh=mesh,
        out_type=jax.ShapeDtypeStruct((NEDGE, 128), F32),
        scratch_types=[
            pltpu.VMEM((CH,), jnp.int32),
            pltpu.VMEM((CH, 128), F32),
            pltpu.SemaphoreType.DMA,
        ],
    )
    def k_fn(v1_hbm, idx_hbm, g1_hbm, idx_v, rows_v, sem):
        wid = lax.axis_index("s") * 2 + lax.axis_index("c")
        i0 = wid * npts

        def body(t, carry):
            eoff = (t // 2) * N + i0 + (t % 2) * CH
            pltpu.sync_copy(idx_hbm.at[pl.ds(eoff, CH)], idx_v)
            pltpu.async_copy(v1_hbm.at[idx_v], rows_v, sem).wait()
            pltpu.sync_copy(rows_v, g1_hbm.at[pl.ds(eoff, CH)])
            return carry

        lax.fori_loop(0, K * (npts // CH), body, 0)

    return k_fn(V1, idxf)


def _sc_maxgather(V2, idxf):
    mesh = plsc.VectorSubcoreMesh(core_axis_name="c", subcore_axis_name="s")
    CH = 128
    npts = N // 32

    @functools.partial(
        pl.kernel,
        mesh=mesh,
        out_type=jax.ShapeDtypeStruct((N, 128), F32),
        scratch_types=[
            pltpu.VMEM((CH,), jnp.int32),
            pltpu.VMEM((CH, 128), F32),
            pltpu.VMEM((CH, 128), F32),
            pltpu.SemaphoreType.DMA,
        ],
    )
    def k_fn(v2_hbm, idx_hbm, m_hbm, idx_v, rows_v, acc_v, sem):
        wid = lax.axis_index("s") * 2 + lax.axis_index("c")
        i0 = wid * npts

        def half(h, carry):
            base = i0 + h * CH

            def plane(k, c2):
                pltpu.sync_copy(idx_hbm.at[pl.ds(k * N + base, CH)], idx_v)
                @pl.when(k == 0)
                def _():
                    pltpu.async_copy(v2_hbm.at[idx_v], acc_v, sem).wait()

                @pl.when(k > 0)
                def _():
                    pltpu.async_copy(v2_hbm.at[idx_v], rows_v, sem).wait()

                    def row(r, c3):
                        for c in range(8):
                            sl = pl.ds(c * 16, 16)
                            acc_v[r, sl] = jnp.maximum(
                                acc_v[r, sl], rows_v[r, sl])
                        return c3

                    lax.fori_loop(0, CH, row, 0)
                return c2

            lax.fori_loop(0, K, plane, 0)
            pltpu.sync_copy(acc_v, m_hbm.at[pl.ds(base, CH)])
            return carry

        lax.fori_loop(0, npts // CH, half, 0)

    return k_fn(V2, idxf)


def _b_body(g1_ref, pos_ref, w1a_ref, w1b_ref, b1_ref, g_ref, bt_ref,
            w2_ref, b2_ref, h2_ref, st2_ref, st1_s):
    p = pl.program_id(0)
    k = pl.program_id(1)
    i = pl.program_id(2)
    first = (k == 0) & (i == 0)
    xi = pos_ref[...]
    h1 = (_dot(xi, w1a_ref[...])
          + _dot(g1_ref[:, :8] - xi, w1b_ref[...]) + b1_ref[...])

    @pl.when((p == 0) & first)
    def _():
        st1_s[...] = jnp.zeros_like(st1_s)

    @pl.when(p == 0)
    def _():
        st1_s[0:1, :] += jnp.sum(h1, axis=0, keepdims=True)
        st1_s[1:2, :] += jnp.sum(h1 * h1, axis=0, keepdims=True)

    @pl.when((p == 1) & first)
    def _():
        st2_ref[...] = jnp.zeros_like(st2_ref)

    @pl.when(p == 1)
    def _():
        mtot = jnp.float32(NEDGE)
        mean = st1_s[0:1, :] / mtot
        var = st1_s[1:2, :] / mtot - mean * mean
        al = g_ref[...] * lax.rsqrt(var + 1e-5)
        be = bt_ref[...] - al * mean
        y1 = jnp.maximum(al * h1 + be, 0.0)
        h2 = _dot(y1, w2_ref[...]) + b2_ref[...]
        h2_ref[...] = h2
        st2_ref[0:1, :] += jnp.sum(h2, axis=0, keepdims=True)
        st2_ref[1:2, :] += jnp.sum(h2 * h2, axis=0, keepdims=True)


def _conv1_mid(G1, posp, w1a, w1b, b1r, g1r, bt1r, W2, b2r):
    BR = 512
    nb = N // BR
    return pl.pallas_call(
        _b_body,
        grid=(2, K, nb),
        in_specs=[
            pl.BlockSpec((BR, 128), lambda p, k, i: (k * nb + i, 0)),
            pl.BlockSpec((BR, 8), lambda p, k, i: (i, 0)),
            pl.BlockSpec((8, 64), lambda p, k, i: (0, 0)),
            pl.BlockSpec((8, 64), lambda p, k, i: (0, 0)),
            pl.BlockSpec((1, 64), lambda p, k, i: (0, 0)),
            pl.BlockSpec((1, 64), lambda p, k, i: (0, 0)),
            pl.BlockSpec((1, 64), lambda p, k, i: (0, 0)),
            pl.BlockSpec((64, 64), lambda p, k, i: (0, 0)),
            pl.BlockSpec((1, 64), lambda p, k, i: (0, 0)),
        ],
        out_specs=[
            pl.BlockSpec((BR, 64), lambda p, k, i: (k * nb + i, 0)),
            pl.BlockSpec((2, 64), lambda p, k, i: (0, 0)),
        ],
        out_shape=[
            jax.ShapeDtypeStruct((NEDGE, 64), F32),
            jax.ShapeDtypeStruct((2, 64), F32),
        ],
        scratch_shapes=[pltpu.VMEM((2, 64), F32)],
    )(G1, posp, w1a, w1b, b1r, g1r, bt1r, W2, b2r)


def _c_body(h2_ref, st2_ref, g_ref, bt_ref, w3_ref, b3_ref,
            wd4_ref, w4b_ref, b4_ref, x1_ref, a2_ref, v2_ref, acc_s):
    k = pl.program_id(1)
    mtot = jnp.float32(NEDGE)
    mean = st2_ref[0:1, :] / mtot
    var = st2_ref[1:2, :] / mtot - mean * mean
    al = g_ref[...] * lax.rsqrt(var + 1e-5)
    be = bt_ref[...] - al * mean
    y2 = jnp.maximum(al * h2_ref[...] + be, 0.0)
    h3 = _dot(y2, w3_ref[...]) + b3_ref[...]

    @pl.when(k == 0)
    def _():
        acc_s[...] = h3

    @pl.when(k > 0)
    def _():
        acc_s[...] = jnp.maximum(acc_s[...], h3)

    @pl.when(k == K - 1)
    def _():
        x1b = acc_s[...]
        x1_ref[...] = x1b
        a2_ref[...] = _dot(x1b, wd4_ref[...]) + b4_ref[...]
        v2_ref[...] = _dot(x1b, w4b_ref[...])


def _conv1_tail(h2, st2, g2r, bt2r, W3, b3r, wd4, w4b, b4r):
    BR = 512
    nb = N // BR
    return pl.pallas_call(
        _c_body,
        grid=(nb, K),
        in_specs=[
            pl.BlockSpec((BR, 64), lambda i, k: (k * nb + i, 0)),
            pl.BlockSpec((2, 64), lambda i, k: (0, 0)),
            pl.BlockSpec((1, 64), lambda i, k: (0, 0)),
            pl.BlockSpec((1, 64), lambda i, k: (0, 0)),
            pl.BlockSpec((64, 64), lambda i, k: (0, 0)),
            pl.BlockSpec((1, 64), lambda i, k: (0, 0)),
            pl.BlockSpec((64, 128), lambda i, k: (0, 0)),
            pl.BlockSpec((64, 128), lambda i, k: (0, 0)),
            pl.BlockSpec((1, 128), lambda i, k: (0, 0)),
        ],
        out_specs=[
            pl.BlockSpec((BR, 64), lambda i, k: (i, 0)),
            pl.BlockSpec((BR, 128), lambda i, k: (i, 0)),
            pl.BlockSpec((BR, 128), lambda i, k: (i, 0)),
        ],
        out_shape=[
            jax.ShapeDtypeStruct((N, 64), F32),
            jax.ShapeDtypeStruct((N, 128), F32),
            jax.ShapeDtypeStruct((N, 128), F32),
        ],
        scratch_shapes=[pltpu.VMEM((BR, 64), F32)],
    )(h2, st2, g2r, bt2r, W3, b3r, wd4, w4b, b4r)


def _e_body(x1_ref, a2_ref, m_ref, b_ref, w5a_ref, w5b_ref, b5_ref,
            w6_ref, b6_ref, w7_ref, b7_ref, w8_ref, b8_ref,
            out_ref, pool_s, *, BR, NB):
    i = pl.program_id(0)

    @pl.when(i == 0)
    def _():
        pool_s[...] = jnp.full_like(pool_s, -jnp.inf)

    x2 = a2_ref[...] + m_ref[...]
    o = _dot(x1_ref[...], w5a_ref[...]) + _dot(x2, w5b_ref[...]) + b5_ref[...]
    bcol = b_ref[...]
    bmin = bcol[0, 0]
    bmax = bcol[BR - 1, 0]
    for s in range(NUM_GRAPHS):
        @pl.when((bmin <= s) & (s <= bmax))
        def _():
            seg = jnp.where(bcol == s, o, -jnp.inf)
            pool_s[s:s + 1, :] = jnp.maximum(
                pool_s[s:s + 1, :],
                jnp.max(seg, axis=0, keepdims=True))

    @pl.when(i == NB - 1)
    def _():
        h = jnp.maximum(_dot(pool_s[...], w6_ref[...]) + b6_ref[...], 0.0)
        h = jnp.maximum(_dot(h, w7_ref[...]) + b7_ref[...], 0.0)
        out_ref[...] = _dot(h, w8_ref[...]) + b8_ref[...]


def _tail(x1, A2, M, bcol, w5a, w5b, b5r, W6, b6r, W7, b7r, W8, b8r):
    BR = 512
    NB = N // BR
    return pl.pallas_call(
        functools.partial(_e_body, BR=BR, NB=NB),
        grid=(NB,),
        in_specs=[
            pl.BlockSpec((BR, 64), lambda i: (i, 0)),
            pl.BlockSpec((BR, 128), lambda i: (i, 0)),
            pl.BlockSpec((BR, 128), lambda i: (i, 0)),
            pl.BlockSpec((BR, 1), lambda i: (i, 0)),
            pl.BlockSpec((64, 1024), lambda i: (0, 0)),
            pl.BlockSpec((128, 1024), lambda i: (0, 0)),
            pl.BlockSpec((1, 1024), lambda i: (0, 0)),
            pl.BlockSpec((1024, 512), lambda i: (0, 0)),
            pl.BlockSpec((1, 512), lambda i: (0, 0)),
            pl.BlockSpec((512, 256), lambda i: (0, 0)),
            pl.BlockSpec((1, 256), lambda i: (0, 0)),
            pl.BlockSpec((256, 40), lambda i: (0, 0)),
            pl.BlockSpec((1, 40), lambda i: (0, 0)),
        ],
        out_specs=pl.BlockSpec((NUM_GRAPHS, 40), lambda i: (0, 0)),
        out_shape=jax.ShapeDtypeStruct((NUM_GRAPHS, 40), F32),
        scratch_shapes=[pltpu.VMEM((NUM_GRAPHS, 1024), F32)],
    )(x1, A2, M, bcol, w5a, w5b, b5r, W6, b6r, W7, b7r, W8, b8r)


def kernel(pos, batch, W1, b1, g1, bt1, W2, b2, g2, bt2, W3, b3,
           W4, b4, W5, b5, W6, b6, W7, b7, W8, b8):
    batch = batch.astype(jnp.int32)
    br = batch[:, None]
    NT = N // 512
    bc3 = batch.reshape(NT, 1, 512)

    def _tiles(xt):
        return xt.reshape(xt.shape[0], NT, 512).transpose(1, 0, 2)

    posp = jnp.pad(pos, ((0, 0), (0, 5)))
    posp128 = jnp.pad(pos, ((0, 0), (0, 125)))
    w1a = jnp.pad(W1[:3], ((0, 5), (0, 0)))
    w1b = jnp.pad(W1[3:], ((0, 5), (0, 0)))

    idx1 = _knn(posp, _tiles(posp.T), br, bc3)
    G1 = _sc_gather(posp128, idx1.T.reshape(-1))

    h2, st2 = _conv1_mid(G1, posp, w1a, w1b, b1[None, :],
                         g1[None, :], bt1[None, :], W2, b2[None, :])
    x1, A2, V2 = _conv1_tail(
        h2, st2, g2[None, :], bt2[None, :], W3, b3[None, :],
        W4[:64] - W4[64:], W4[64:], b4[None, :])

    idx2 = _knn(x1, _tiles(x1.T), br, bc3)
    M = _sc_maxgather(V2, idx2.T.reshape(-1))

    return _tail(x1, A2, M, br, W5[:64], W5[64:], b5[None, :],
                 W6, b6[None, :], W7, b7[None, :], W8, b8[None, :])


# speedup vs baseline: 4.0883x; 4.0883x over previous
import functools

import jax
import jax.numpy as jnp
from jax import lax
from jax.experimental import pallas as pl
from jax.experimental.pallas import tpu as pltpu
from jax.experimental.pallas import tpu_sc as plsc

N = 8192
K = 20
NUM_GRAPHS = 8
F32 = jnp.float32
HIGH = lax.Precision.HIGHEST
MASKV = 1e37
NEDGE = N * K


def _dot(a, b):
    return lax.dot(a, b, precision=lax.Precision.DEFAULT,
                   preferred_element_type=F32)


def _extract_topk(dist_s, idx_ref, BR, W, base):
    colids = lax.broadcasted_iota(jnp.int32, (BR, W), 1)
    cols = []
    for _ in range(K):
        dcur = dist_s[...]
        m = jnp.min(dcur, axis=1, keepdims=True)
        cand = jnp.where(dcur == m, colids, jnp.int32(N))
        j = jnp.min(cand, axis=1, keepdims=True)
        cols.append(jnp.minimum(j + base, N - 1))
        dist_s[...] = jnp.where(colids == j, jnp.inf, dcur)
    idx_ref[...] = jnp.concatenate(cols, axis=1)


def _knn_full_body(x_ref, xt_ref, br_ref, bc_ref, idx_ref, dist_s, *, BR):
    xb = x_ref[...]
    sqr = jnp.sum(xb * xb, axis=1, keepdims=True)
    xt = xt_ref[...]
    sqc = jnp.sum(xt * xt, axis=0, keepdims=True)
    d = sqr + sqc - 2.0 * _dot(xb, xt)
    mask = br_ref[...] != bc_ref[...]
    dist_s[...] = jnp.where(mask, MASKV, d)
    _extract_topk(dist_s, idx_ref, BR, N, 0)


def _knn_full(x, xt, br, bc):
    BR = 64
    f = x.shape[1]
    return pl.pallas_call(
        functools.partial(_knn_full_body, BR=BR),
        grid=(N // BR,),
        in_specs=[
            pl.BlockSpec((BR, f), lambda i: (i, 0)),
            pl.BlockSpec((f, N), lambda i: (0, 0)),
            pl.BlockSpec((BR, 1), lambda i: (i, 0)),
            pl.BlockSpec((1, N), lambda i: (0, 0)),
        ],
        out_specs=pl.BlockSpec((BR, K), lambda i: (i, 0)),
        out_shape=jax.ShapeDtypeStruct((N, K), jnp.int32),
        scratch_shapes=[pltpu.VMEM((BR, N), F32)],
    )(x, xt, br, bc)


SLOT = 1536
NP = NUM_GRAPHS * SLOT


def _knn_pad_body(x_ref, xtg_ref, br_ref, bcg_ref, st_ref, idx_ref, dist_s,
                  *, BR):
    xb = x_ref[...]
    sqr = jnp.sum(xb * xb, axis=1, keepdims=True)
    xt = xtg_ref[0]
    sqc = jnp.sum(xt * xt, axis=0, keepdims=True)
    d = sqr + sqc - 2.0 * _dot(xb, xt)
    mask = br_ref[...] != bcg_ref[0]
    dist_s[...] = jnp.where(mask, MASKV, d)
    _extract_topk(dist_s, idx_ref, BR, SLOT, st_ref[0, 0, 0])


def _knn_pad(xp, xtg, brp, bcg, starts):
    BR = 128
    bpg = SLOT // BR
    f = xp.shape[1]
    return pl.pallas_call(
        functools.partial(_knn_pad_body, BR=BR),
        grid=(NP // BR,),
        in_specs=[
            pl.BlockSpec((BR, f), lambda i: (i, 0)),
            pl.BlockSpec((1, f, SLOT), lambda i: (i // bpg, 0, 0)),
            pl.BlockSpec((BR, 1), lambda i: (i, 0)),
            pl.BlockSpec((1, 1, SLOT), lambda i: (i // bpg, 0, 0)),
            pl.BlockSpec((1, 1, 1), lambda i: (i // bpg, 0, 0)),
        ],
        out_specs=pl.BlockSpec((BR, K), lambda i: (i, 0)),
        out_shape=jax.ShapeDtypeStruct((NP, K), jnp.int32),
        scratch_shapes=[pltpu.VMEM((BR, SLOT), F32)],
    )(xp, xtg, brp, bcg, starts)


def _sc_gather(V1, idxf):
    mesh = plsc.VectorSubcoreMesh(core_axis_name="c", subcore_axis_name="s")
    CH = 128
    E = idxf.shape[0]
    chunk = E // 32
    dt = V1.dtype

    @functools.partial(
        pl.kernel,
        mesh=mesh,
        out_type=jax.ShapeDtypeStruct((E, 128), dt),
        scratch_types=[
            pltpu.VMEM((CH,), jnp.int32),
            pltpu.VMEM((CH, 128), dt),
            pltpu.SemaphoreType.DMA,
        ],
    )
    def k_fn(v1_hbm, idx_hbm, g1_hbm, idx_v, rows_v, sem):
        wid = lax.axis_index("s") * 2 + lax.axis_index("c")
        i0 = wid * chunk

        def body(t, carry):
            eoff = i0 + t * CH
            pltpu.sync_copy(idx_hbm.at[pl.ds(eoff, CH)], idx_v)
            pltpu.async_copy(v1_hbm.at[idx_v], rows_v, sem).wait()
            pltpu.sync_copy(rows_v, g1_hbm.at[pl.ds(eoff, CH)])
            return carry

        lax.fori_loop(0, chunk // CH, body, 0)

    return k_fn(V1, idxf)


def _sc_maxgather(V2, idxf):
    mesh = plsc.VectorSubcoreMesh(core_axis_name="c", subcore_axis_name="s")
    CH = 128
    npts = N // 32

    @functools.partial(
        pl.kernel,
        mesh=mesh,
        out_type=jax.ShapeDtypeStruct((N, 128), F32),
        scratch_types=[
            pltpu.VMEM((CH,), jnp.int32),
            pltpu.VMEM((CH, 128), F32),
            pltpu.VMEM((CH, 128), F32),
            pltpu.SemaphoreType.DMA,
        ],
    )
    def k_fn(v2_hbm, idx_hbm, m_hbm, idx_v, rows_v, acc_v, sem):
        wid = lax.axis_index("s") * 2 + lax.axis_index("c")
        i0 = wid * npts

        def half(h, carry):
            base = i0 + h * CH

            def plane(k, c2):
                pltpu.sync_copy(idx_hbm.at[pl.ds(k * N + base, CH)], idx_v)
                @pl.when(k == 0)
                def _():
                    pltpu.async_copy(v2_hbm.at[idx_v], acc_v, sem).wait()

                @pl.when(k > 0)
                def _():
                    pltpu.async_copy(v2_hbm.at[idx_v], rows_v, sem).wait()

                    def row(r, c3):
                        for c in range(8):
                            sl = pl.ds(c * 16, 16)
                            acc_v[r, sl] = jnp.maximum(
                                acc_v[r, sl], rows_v[r, sl])
                        return c3

                    lax.fori_loop(0, CH, row, 0)
                return c2

            lax.fori_loop(0, K, plane, 0)
            pltpu.sync_copy(acc_v, m_hbm.at[pl.ds(base, CH)])
            return carry

        lax.fori_loop(0, npts // CH, half, 0)

    return k_fn(V2, idxf)


def _b_body(g1_ref, pos_ref, w1a_ref, w1b_ref, b1_ref, g_ref, bt_ref,
            w2_ref, b2_ref, h2_ref, st2_ref, st1_s):
    p = pl.program_id(0)
    k = pl.program_id(1)
    i = pl.program_id(2)
    first = (k == 0) & (i == 0)
    xi = pos_ref[...]
    h1 = (_dot(xi, w1a_ref[...])
          + _dot(g1_ref[:, :8] - xi, w1b_ref[...]) + b1_ref[...])

    @pl.when((p == 0) & first)
    def _():
        st1_s[...] = jnp.zeros_like(st1_s)

    @pl.when(p == 0)
    def _():
        st1_s[0:1, :] += jnp.sum(h1, axis=0, keepdims=True)
        st1_s[1:2, :] += jnp.sum(h1 * h1, axis=0, keepdims=True)

    @pl.when((p == 1) & first)
    def _():
        st2_ref[...] = jnp.zeros_like(st2_ref)

    @pl.when(p == 1)
    def _():
        mtot = jnp.float32(NEDGE)
        mean = st1_s[0:1, :] / mtot
        var = st1_s[1:2, :] / mtot - mean * mean
        al = g_ref[...] * lax.rsqrt(var + 1e-5)
        be = bt_ref[...] - al * mean
        y1 = jnp.maximum(al * h1 + be, 0.0)
        h2 = _dot(y1, w2_ref[...]) + b2_ref[...]
        h2_ref[...] = h2
        st2_ref[0:1, :] += jnp.sum(h2, axis=0, keepdims=True)
        st2_ref[1:2, :] += jnp.sum(h2 * h2, axis=0, keepdims=True)


def _conv1_mid(G1, posp, w1a, w1b, b1r, g1r, bt1r, W2, b2r):
    BR = 512
    nb = N // BR
    return pl.pallas_call(
        _b_body,
        grid=(2, K, nb),
        in_specs=[
            pl.BlockSpec((BR, 128), lambda p, k, i: (k * nb + i, 0)),
            pl.BlockSpec((BR, 8), lambda p, k, i: (i, 0)),
            pl.BlockSpec((8, 64), lambda p, k, i: (0, 0)),
            pl.BlockSpec((8, 64), lambda p, k, i: (0, 0)),
            pl.BlockSpec((1, 64), lambda p, k, i: (0, 0)),
            pl.BlockSpec((1, 64), lambda p, k, i: (0, 0)),
            pl.BlockSpec((1, 64), lambda p, k, i: (0, 0)),
            pl.BlockSpec((64, 64), lambda p, k, i: (0, 0)),
            pl.BlockSpec((1, 64), lambda p, k, i: (0, 0)),
        ],
        out_specs=[
            pl.BlockSpec((BR, 64), lambda p, k, i: (k * nb + i, 0)),
            pl.BlockSpec((2, 64), lambda p, k, i: (0, 0)),
        ],
        out_shape=[
            jax.ShapeDtypeStruct((NEDGE, 64), F32),
            jax.ShapeDtypeStruct((2, 64), F32),
        ],
        scratch_shapes=[pltpu.VMEM((2, 64), F32)],
    )(G1, posp, w1a, w1b, b1r, g1r, bt1r, W2, b2r)


def _c_body(h2_ref, st2_ref, g_ref, bt_ref, w3_ref, b3_ref,
            wd4_ref, w4b_ref, b4_ref, x1_ref, a2_ref, v2_ref, acc_s):
    k = pl.program_id(1)
    mtot = jnp.float32(NEDGE)
    mean = st2_ref[0:1, :] / mtot
    var = st2_ref[1:2, :] / mtot - mean * mean
    al = g_ref[...] * lax.rsqrt(var + 1e-5)
    be = bt_ref[...] - al * mean
    y2 = jnp.maximum(al * h2_ref[...] + be, 0.0)
    h3 = _dot(y2, w3_ref[...]) + b3_ref[...]

    @pl.when(k == 0)
    def _():
        acc_s[...] = h3

    @pl.when(k > 0)
    def _():
        acc_s[...] = jnp.maximum(acc_s[...], h3)

    @pl.when(k == K - 1)
    def _():
        x1b = acc_s[...]
        x1_ref[...] = x1b
        a2_ref[...] = _dot(x1b, wd4_ref[...]) + b4_ref[...]
        v2_ref[...] = _dot(x1b, w4b_ref[...])


def _conv1_tail(h2, st2, g2r, bt2r, W3, b3r, wd4, w4b, b4r):
    BR = 512
    nb = N // BR
    return pl.pallas_call(
        _c_body,
        grid=(nb, K),
        in_specs=[
            pl.BlockSpec((BR, 64), lambda i, k: (k * nb + i, 0)),
            pl.BlockSpec((2, 64), lambda i, k: (0, 0)),
            pl.BlockSpec((1, 64), lambda i, k: (0, 0)),
            pl.BlockSpec((1, 64), lambda i, k: (0, 0)),
            pl.BlockSpec((64, 64), lambda i, k: (0, 0)),
            pl.BlockSpec((1, 64), lambda i, k: (0, 0)),
            pl.BlockSpec((64, 128), lambda i, k: (0, 0)),
            pl.BlockSpec((64, 128), lambda i, k: (0, 0)),
            pl.BlockSpec((1, 128), lambda i, k: (0, 0)),
        ],
        out_specs=[
            pl.BlockSpec((BR, 64), lambda i, k: (i, 0)),
            pl.BlockSpec((BR, 128), lambda i, k: (i, 0)),
            pl.BlockSpec((BR, 128), lambda i, k: (i, 0)),
        ],
        out_shape=[
            jax.ShapeDtypeStruct((N, 64), F32),
            jax.ShapeDtypeStruct((N, 128), F32),
            jax.ShapeDtypeStruct((N, 128), F32),
        ],
        scratch_shapes=[pltpu.VMEM((BR, 64), F32)],
    )(h2, st2, g2r, bt2r, W3, b3r, wd4, w4b, b4r)


def _e_body(x1_ref, a2_ref, m_ref, b_ref, w5a_ref, w5b_ref, b5_ref,
            w6_ref, b6_ref, w7_ref, b7_ref, w8_ref, b8_ref,
            out_ref, pool_s, *, BR, NB):
    i = pl.program_id(0)

    @pl.when(i == 0)
    def _():
        pool_s[...] = jnp.full_like(pool_s, -jnp.inf)

    x2 = a2_ref[...] + m_ref[...]
    o = _dot(x1_ref[...], w5a_ref[...]) + _dot(x2, w5b_ref[...]) + b5_ref[...]
    bcol = b_ref[...]
    bmin = bcol[0, 0]
    bmax = bcol[BR - 1, 0]
    for s in range(NUM_GRAPHS):
        @pl.when((bmin <= s) & (s <= bmax))
        def _():
            seg = jnp.where(bcol == s, o, -jnp.inf)
            pool_s[s:s + 1, :] = jnp.maximum(
                pool_s[s:s + 1, :],
                jnp.max(seg, axis=0, keepdims=True))

    @pl.when(i == NB - 1)
    def _():
        h = jnp.maximum(_dot(pool_s[...], w6_ref[...]) + b6_ref[...], 0.0)
        h = jnp.maximum(_dot(h, w7_ref[...]) + b7_ref[...], 0.0)
        out_ref[...] = _dot(h, w8_ref[...]) + b8_ref[...]


def _tail(x1, A2, M, bcol, w5a, w5b, b5r, W6, b6r, W7, b7r, W8, b8r):
    BR = 512
    NB = N // BR
    return pl.pallas_call(
        functools.partial(_e_body, BR=BR, NB=NB),
        grid=(NB,),
        in_specs=[
            pl.BlockSpec((BR, 64), lambda i: (i, 0)),
            pl.BlockSpec((BR, 128), lambda i: (i, 0)),
            pl.BlockSpec((BR, 128), lambda i: (i, 0)),
            pl.BlockSpec((BR, 1), lambda i: (i, 0)),
            pl.BlockSpec((64, 1024), lambda i: (0, 0)),
            pl.BlockSpec((128, 1024), lambda i: (0, 0)),
            pl.BlockSpec((1, 1024), lambda i: (0, 0)),
            pl.BlockSpec((1024, 512), lambda i: (0, 0)),
            pl.BlockSpec((1, 512), lambda i: (0, 0)),
            pl.BlockSpec((512, 256), lambda i: (0, 0)),
            pl.BlockSpec((1, 256), lambda i: (0, 0)),
            pl.BlockSpec((256, 40), lambda i: (0, 0)),
            pl.BlockSpec((1, 40), lambda i: (0, 0)),
        ],
        out_specs=pl.BlockSpec((NUM_GRAPHS, 40), lambda i: (0, 0)),
        out_shape=jax.ShapeDtypeStruct((NUM_GRAPHS, 40), F32),
        scratch_shapes=[pltpu.VMEM((NUM_GRAPHS, 1024), F32)],
    )(x1, A2, M, bcol, w5a, w5b, b5r, W6, b6r, W7, b7r, W8, b8r)


def kernel(pos, batch, W1, b1, g1, bt1, W2, b2, g2, bt2, W3, b3,
           W4, b4, W5, b5, W6, b6, W7, b7, W8, b8):
    batch = batch.astype(jnp.int32)
    br = batch[:, None]
    bc = batch[None, :]

    gids = jnp.arange(NUM_GRAPHS, dtype=jnp.int32)
    starts = jnp.searchsorted(batch, gids).astype(jnp.int32)
    ends = jnp.searchsorted(batch, gids, side="right").astype(jnp.int32)
    counts = ends - starts
    fits = jnp.max(counts) <= SLOT
    slot = jnp.arange(NP, dtype=jnp.int32)
    sg = slot // SLOT
    soff = slot % SLOT
    p = jnp.minimum(starts[sg] + soff, N - 1)
    batch_pad = jnp.where(soff < counts[sg], sg, -1).astype(jnp.int32)
    brp = batch_pad[:, None]
    bcg = batch_pad.reshape(NUM_GRAPHS, 1, SLOT)
    starts2 = starts.reshape(NUM_GRAPHS, 1, 1)
    pinv = batch * SLOT + (jnp.arange(N, dtype=jnp.int32) - starts[batch])

    def _knn_via_slots(x128, f):
        xp = _sc_gather(x128, p)[:, :f]
        xtg = xp.T.reshape(f, NUM_GRAPHS, SLOT).transpose(1, 0, 2)
        idxp = _knn_pad(xp, xtg, brp, bcg, starts2)
        idxp128 = jnp.pad(idxp, ((0, 0), (0, 128 - K)))
        return _sc_gather(idxp128, pinv)[:, :K]

    posp = jnp.pad(pos, ((0, 0), (0, 5)))
    posp128 = jnp.pad(pos, ((0, 0), (0, 125)))
    w1a = jnp.pad(W1[:3], ((0, 5), (0, 0)))
    w1b = jnp.pad(W1[3:], ((0, 5), (0, 0)))

    idx1 = lax.cond(fits,
                    lambda: _knn_via_slots(posp128, 8),
                    lambda: _knn_full(posp, posp.T, br, bc))
    G1 = _sc_gather(posp128, idx1.T.reshape(-1))

    h2, st2 = _conv1_mid(G1, posp, w1a, w1b, b1[None, :],
                         g1[None, :], bt1[None, :], W2, b2[None, :])
    x1, A2, V2 = _conv1_tail(
        h2, st2, g2[None, :], bt2[None, :], W3, b3[None, :],
        W4[:64] - W4[64:], W4[64:], b4[None, :])

    x1p128 = jnp.pad(x1, ((0, 0), (0, 64)))
    idx2 = lax.cond(fits,
                    lambda: _knn_via_slots(x1p128, 64),
                    lambda: _knn_full(x1, x1.T, br, bc))
    M = _sc_maxgather(V2, idx2.T.reshape(-1))

    return _tail(x1, A2, M, br, W5[:64], W5[64:], b5[None, :],
                 W6, b6[None, :], W7, b7[None, :], W8, b8[None, :])


# speedup vs baseline: 5.1926x; 1.2701x over previous
import functools

import jax
import jax.numpy as jnp
from jax import lax
from jax.experimental import pallas as pl
from jax.experimental.pallas import tpu as pltpu
from jax.experimental.pallas import tpu_sc as plsc

N = 8192
K = 20
NUM_GRAPHS = 8
F32 = jnp.float32
HIGH = lax.Precision.HIGHEST
MASKV = 1e37
NEDGE = N * K


def _dot(a, b):
    return lax.dot(a, b, precision=lax.Precision.DEFAULT,
                   preferred_element_type=F32)


def _extract_topk(dist_s, idx_ref, BR, W, base):
    colids = lax.broadcasted_iota(jnp.int32, (BR, W), 1)
    cols = []
    for _ in range(K):
        dcur = dist_s[...]
        m = jnp.min(dcur, axis=1, keepdims=True)
        cand = jnp.where(dcur == m, colids, jnp.int32(N))
        j = jnp.min(cand, axis=1, keepdims=True)
        cols.append(jnp.minimum(j + base, N - 1))
        dist_s[...] = jnp.where(colids == j, jnp.inf, dcur)
    idx_ref[...] = jnp.concatenate(cols, axis=1)


def _knn_full_body(x_ref, xt_ref, br_ref, bc_ref, idx_ref, dist_s, *, BR):
    xb = x_ref[...]
    sqr = jnp.sum(xb * xb, axis=1, keepdims=True)
    xt = xt_ref[...]
    sqc = jnp.sum(xt * xt, axis=0, keepdims=True)
    d = sqr + sqc - 2.0 * _dot(xb, xt)
    mask = br_ref[...] != bc_ref[...]
    dist_s[...] = jnp.where(mask, MASKV, d)
    _extract_topk(dist_s, idx_ref, BR, N, 0)


def _knn_full(x, xt, br, bc):
    BR = 64
    f = x.shape[1]
    return pl.pallas_call(
        functools.partial(_knn_full_body, BR=BR),
        grid=(N // BR,),
        in_specs=[
            pl.BlockSpec((BR, f), lambda i: (i, 0)),
            pl.BlockSpec((f, N), lambda i: (0, 0)),
            pl.BlockSpec((BR, 1), lambda i: (i, 0)),
            pl.BlockSpec((1, N), lambda i: (0, 0)),
        ],
        out_specs=pl.BlockSpec((BR, K), lambda i: (i, 0)),
        out_shape=jax.ShapeDtypeStruct((N, K), jnp.int32),
        scratch_shapes=[pltpu.VMEM((BR, N), F32)],
    )(x, xt, br, bc)


SLOT = 1280
NP = NUM_GRAPHS * SLOT


def _knn_pad_body(x_ref, xtg_ref, br_ref, bcg_ref, st_ref, idx_ref, dist_s,
                  *, BR):
    xb = x_ref[...]
    sqr = jnp.sum(xb * xb, axis=1, keepdims=True)
    xt = xtg_ref[0]
    sqc = jnp.sum(xt * xt, axis=0, keepdims=True)
    d = sqr + sqc - 2.0 * _dot(xb, xt)
    mask = br_ref[...] != bcg_ref[0]
    dist_s[...] = jnp.where(mask, MASKV, d)
    _extract_topk(dist_s, idx_ref, BR, SLOT, st_ref[0, 0, 0])


def _knn_pad(xp, xtg, brp, bcg, starts):
    BR = 256
    bpg = SLOT // BR
    f = xp.shape[1]
    return pl.pallas_call(
        functools.partial(_knn_pad_body, BR=BR),
        grid=(NP // BR,),
        in_specs=[
            pl.BlockSpec((BR, f), lambda i: (i, 0)),
            pl.BlockSpec((1, f, SLOT), lambda i: (i // bpg, 0, 0)),
            pl.BlockSpec((BR, 1), lambda i: (i, 0)),
            pl.BlockSpec((1, 1, SLOT), lambda i: (i // bpg, 0, 0)),
            pl.BlockSpec((1, 1, 1), lambda i: (i // bpg, 0, 0)),
        ],
        out_specs=pl.BlockSpec((BR, K), lambda i: (i, 0)),
        out_shape=jax.ShapeDtypeStruct((NP, K), jnp.int32),
        scratch_shapes=[pltpu.VMEM((BR, SLOT), F32)],
    )(xp, xtg, brp, bcg, starts)


def _sc_gather(V1, idxf):
    mesh = plsc.VectorSubcoreMesh(core_axis_name="c", subcore_axis_name="s")
    E = idxf.shape[0]
    chunk = E // 32
    CH = 128 if chunk % 128 == 0 else 64
    dt = V1.dtype

    @functools.partial(
        pl.kernel,
        mesh=mesh,
        out_type=jax.ShapeDtypeStruct((E, 128), dt),
        scratch_types=[
            pltpu.VMEM((CH,), jnp.int32),
            pltpu.VMEM((CH, 128), dt),
            pltpu.SemaphoreType.DMA,
        ],
    )
    def k_fn(v1_hbm, idx_hbm, g1_hbm, idx_v, rows_v, sem):
        wid = lax.axis_index("s") * 2 + lax.axis_index("c")
        i0 = wid * chunk

        def body(t, carry):
            eoff = i0 + t * CH
            pltpu.sync_copy(idx_hbm.at[pl.ds(eoff, CH)], idx_v)
            pltpu.async_copy(v1_hbm.at[idx_v], rows_v, sem).wait()
            pltpu.sync_copy(rows_v, g1_hbm.at[pl.ds(eoff, CH)])
            return carry

        lax.fori_loop(0, chunk // CH, body, 0)

    return k_fn(V1, idxf)


def _sc_maxgather(V2, idxf):
    mesh = plsc.VectorSubcoreMesh(core_axis_name="c", subcore_axis_name="s")
    CH = 128
    npts = N // 32

    @functools.partial(
        pl.kernel,
        mesh=mesh,
        out_type=jax.ShapeDtypeStruct((N, 128), F32),
        scratch_types=[
            pltpu.VMEM((CH,), jnp.int32),
            pltpu.VMEM((CH, 128), F32),
            pltpu.VMEM((CH, 128), F32),
            pltpu.SemaphoreType.DMA,
        ],
    )
    def k_fn(v2_hbm, idx_hbm, m_hbm, idx_v, rows_v, acc_v, sem):
        wid = lax.axis_index("s") * 2 + lax.axis_index("c")
        i0 = wid * npts

        def half(h, carry):
            base = i0 + h * CH

            def plane(k, c2):
                pltpu.sync_copy(idx_hbm.at[pl.ds(k * N + base, CH)], idx_v)
                @pl.when(k == 0)
                def _():
                    pltpu.async_copy(v2_hbm.at[idx_v], acc_v, sem).wait()

                @pl.when(k > 0)
                def _():
                    pltpu.async_copy(v2_hbm.at[idx_v], rows_v, sem).wait()

                    def row(r, c3):
                        for c in range(8):
                            sl = pl.ds(c * 16, 16)
                            acc_v[r, sl] = jnp.maximum(
                                acc_v[r, sl], rows_v[r, sl])
                        return c3

                    lax.fori_loop(0, CH, row, 0)
                return c2

            lax.fori_loop(0, K, plane, 0)
            pltpu.sync_copy(acc_v, m_hbm.at[pl.ds(base, CH)])
            return carry

        lax.fori_loop(0, npts // CH, half, 0)

    return k_fn(V2, idxf)


def _b_body(g1_ref, pos_ref, w1a_ref, w1b_ref, b1_ref, g_ref, bt_ref,
            w2_ref, b2_ref, h2_ref, st2_ref, st1_s):
    p = pl.program_id(0)
    k = pl.program_id(1)
    i = pl.program_id(2)
    first = (k == 0) & (i == 0)
    xi = pos_ref[...]
    h1 = (_dot(xi, w1a_ref[...])
          + _dot(g1_ref[:, :8] - xi, w1b_ref[...]) + b1_ref[...])

    @pl.when((p == 0) & first)
    def _():
        st1_s[...] = jnp.zeros_like(st1_s)

    @pl.when(p == 0)
    def _():
        st1_s[0:1, :] += jnp.sum(h1, axis=0, keepdims=True)
        st1_s[1:2, :] += jnp.sum(h1 * h1, axis=0, keepdims=True)

    @pl.when((p == 1) & first)
    def _():
        st2_ref[...] = jnp.zeros_like(st2_ref)

    @pl.when(p == 1)
    def _():
        mtot = jnp.float32(NEDGE)
        mean = st1_s[0:1, :] / mtot
        var = st1_s[1:2, :] / mtot - mean * mean
        al = g_ref[...] * lax.rsqrt(var + 1e-5)
        be = bt_ref[...] - al * mean
        y1 = jnp.maximum(al * h1 + be, 0.0)
        h2 = _dot(y1, w2_ref[...]) + b2_ref[...]
        h2_ref[...] = h2
        st2_ref[0:1, :] += jnp.sum(h2, axis=0, keepdims=True)
        st2_ref[1:2, :] += jnp.sum(h2 * h2, axis=0, keepdims=True)


def _conv1_mid(G1, posp, w1a, w1b, b1r, g1r, bt1r, W2, b2r):
    BR = 512
    nb = N // BR
    return pl.pallas_call(
        _b_body,
        grid=(2, K, nb),
        in_specs=[
            pl.BlockSpec((BR, 128), lambda p, k, i: (k * nb + i, 0)),
            pl.BlockSpec((BR, 8), lambda p, k, i: (i, 0)),
            pl.BlockSpec((8, 64), lambda p, k, i: (0, 0)),
            pl.BlockSpec((8, 64), lambda p, k, i: (0, 0)),
            pl.BlockSpec((1, 64), lambda p, k, i: (0, 0)),
            pl.BlockSpec((1, 64), lambda p, k, i: (0, 0)),
            pl.BlockSpec((1, 64), lambda p, k, i: (0, 0)),
            pl.BlockSpec((64, 64), lambda p, k, i: (0, 0)),
            pl.BlockSpec((1, 64), lambda p, k, i: (0, 0)),
        ],
        out_specs=[
            pl.BlockSpec((BR, 64), lambda p, k, i: (k * nb + i, 0)),
            pl.BlockSpec((2, 64), lambda p, k, i: (0, 0)),
        ],
        out_shape=[
            jax.ShapeDtypeStruct((NEDGE, 64), F32),
            jax.ShapeDtypeStruct((2, 64), F32),
        ],
        scratch_shapes=[pltpu.VMEM((2, 64), F32)],
    )(G1, posp, w1a, w1b, b1r, g1r, bt1r, W2, b2r)


def _c_body(h2_ref, st2_ref, g_ref, bt_ref, w3_ref, b3_ref,
            wd4_ref, w4b_ref, b4_ref, x1_ref, a2_ref, v2_ref, acc_s):
    k = pl.program_id(1)
    mtot = jnp.float32(NEDGE)
    mean = st2_ref[0:1, :] / mtot
    var = st2_ref[1:2, :] / mtot - mean * mean
    al = g_ref[...] * lax.rsqrt(var + 1e-5)
    be = bt_ref[...] - al * mean
    y2 = jnp.maximum(al * h2_ref[...] + be, 0.0)
    h3 = _dot(y2, w3_ref[...]) + b3_ref[...]

    @pl.when(k == 0)
    def _():
        acc_s[...] = h3

    @pl.when(k > 0)
    def _():
        acc_s[...] = jnp.maximum(acc_s[...], h3)

    @pl.when(k == K - 1)
    def _():
        x1b = acc_s[...]
        x1_ref[...] = x1b
        a2_ref[...] = _dot(x1b, wd4_ref[...]) + b4_ref[...]
        v2_ref[...] = _dot(x1b, w4b_ref[...])


def _conv1_tail(h2, st2, g2r, bt2r, W3, b3r, wd4, w4b, b4r):
    BR = 512
    nb = N // BR
    return pl.pallas_call(
        _c_body,
        grid=(nb, K),
        in_specs=[
            pl.BlockSpec((BR, 64), lambda i, k: (k * nb + i, 0)),
            pl.BlockSpec((2, 64), lambda i, k: (0, 0)),
            pl.BlockSpec((1, 64), lambda i, k: (0, 0)),
            pl.BlockSpec((1, 64), lambda i, k: (0, 0)),
            pl.BlockSpec((64, 64), lambda i, k: (0, 0)),
            pl.BlockSpec((1, 64), lambda i, k: (0, 0)),
            pl.BlockSpec((64, 128), lambda i, k: (0, 0)),
            pl.BlockSpec((64, 128), lambda i, k: (0, 0)),
            pl.BlockSpec((1, 128), lambda i, k: (0, 0)),
        ],
        out_specs=[
            pl.BlockSpec((BR, 64), lambda i, k: (i, 0)),
            pl.BlockSpec((BR, 128), lambda i, k: (i, 0)),
            pl.BlockSpec((BR, 128), lambda i, k: (i, 0)),
        ],
        out_shape=[
            jax.ShapeDtypeStruct((N, 64), F32),
            jax.ShapeDtypeStruct((N, 128), F32),
            jax.ShapeDtypeStruct((N, 128), F32),
        ],
        scratch_shapes=[pltpu.VMEM((BR, 64), F32)],
    )(h2, st2, g2r, bt2r, W3, b3r, wd4, w4b, b4r)


def _e_body(x1_ref, a2_ref, m_ref, b_ref, w5a_ref, w5b_ref, b5_ref,
            w6_ref, b6_ref, w7_ref, b7_ref, w8_ref, b8_ref,
            out_ref, pool_s, *, BR, NB):
    i = pl.program_id(0)

    @pl.when(i == 0)
    def _():
        pool_s[...] = jnp.full_like(pool_s, -jnp.inf)

    x2 = a2_ref[...] + m_ref[...]
    o = _dot(x1_ref[...], w5a_ref[...]) + _dot(x2, w5b_ref[...]) + b5_ref[...]
    bcol = b_ref[...]
    bmin = bcol[0, 0]
    bmax = bcol[BR - 1, 0]
    for s in range(NUM_GRAPHS):
        @pl.when((bmin <= s) & (s <= bmax))
        def _():
            seg = jnp.where(bcol == s, o, -jnp.inf)
            pool_s[s:s + 1, :] = jnp.maximum(
                pool_s[s:s + 1, :],
                jnp.max(seg, axis=0, keepdims=True))

    @pl.when(i == NB - 1)
    def _():
        h = jnp.maximum(_dot(pool_s[...], w6_ref[...]) + b6_ref[...], 0.0)
        h = jnp.maximum(_dot(h, w7_ref[...]) + b7_ref[...], 0.0)
        out_ref[...] = _dot(h, w8_ref[...]) + b8_ref[...]


def _tail(x1, A2, M, bcol, w5a, w5b, b5r, W6, b6r, W7, b7r, W8, b8r):
    BR = 512
    NB = N // BR
    return pl.pallas_call(
        functools.partial(_e_body, BR=BR, NB=NB),
        grid=(NB,),
        in_specs=[
            pl.BlockSpec((BR, 64), lambda i: (i, 0)),
            pl.BlockSpec((BR, 128), lambda i: (i, 0)),
            pl.BlockSpec((BR, 128), lambda i: (i, 0)),
            pl.BlockSpec((BR, 1), lambda i: (i, 0)),
            pl.BlockSpec((64, 1024), lambda i: (0, 0)),
            pl.BlockSpec((128, 1024), lambda i: (0, 0)),
            pl.BlockSpec((1, 1024), lambda i: (0, 0)),
            pl.BlockSpec((1024, 512), lambda i: (0, 0)),
            pl.BlockSpec((1, 512), lambda i: (0, 0)),
            pl.BlockSpec((512, 256), lambda i: (0, 0)),
            pl.BlockSpec((1, 256), lambda i: (0, 0)),
            pl.BlockSpec((256, 40), lambda i: (0, 0)),
            pl.BlockSpec((1, 40), lambda i: (0, 0)),
        ],
        out_specs=pl.BlockSpec((NUM_GRAPHS, 40), lambda i: (0, 0)),
        out_shape=jax.ShapeDtypeStruct((NUM_GRAPHS, 40), F32),
        scratch_shapes=[pltpu.VMEM((NUM_GRAPHS, 1024), F32)],
    )(x1, A2, M, bcol, w5a, w5b, b5r, W6, b6r, W7, b7r, W8, b8r)


def kernel(pos, batch, W1, b1, g1, bt1, W2, b2, g2, bt2, W3, b3,
           W4, b4, W5, b5, W6, b6, W7, b7, W8, b8):
    batch = batch.astype(jnp.int32)
    br = batch[:, None]
    bc = batch[None, :]

    gids = jnp.arange(NUM_GRAPHS, dtype=jnp.int32)
    starts = jnp.searchsorted(batch, gids).astype(jnp.int32)
    ends = jnp.searchsorted(batch, gids, side="right").astype(jnp.int32)
    counts = ends - starts
    fits = jnp.max(counts) <= SLOT
    slot = jnp.arange(NP, dtype=jnp.int32)
    sg = slot // SLOT
    soff = slot % SLOT
    p = jnp.minimum(starts[sg] + soff, N - 1)
    batch_pad = jnp.where(soff < counts[sg], sg, -1).astype(jnp.int32)
    brp = batch_pad[:, None]
    bcg = batch_pad.reshape(NUM_GRAPHS, 1, SLOT)
    starts2 = starts.reshape(NUM_GRAPHS, 1, 1)
    pinv = batch * SLOT + (jnp.arange(N, dtype=jnp.int32) - starts[batch])

    def _knn_via_slots(x128, f):
        xp = _sc_gather(x128, p)[:, :f]
        xtg = xp.T.reshape(f, NUM_GRAPHS, SLOT).transpose(1, 0, 2)
        idxp = _knn_pad(xp, xtg, brp, bcg, starts2)
        idxp128 = jnp.pad(idxp, ((0, 0), (0, 128 - K)))
        return _sc_gather(idxp128, pinv)[:, :K]

    posp = jnp.pad(pos, ((0, 0), (0, 5)))
    posp128 = jnp.pad(pos, ((0, 0), (0, 125)))
    w1a = jnp.pad(W1[:3], ((0, 5), (0, 0)))
    w1b = jnp.pad(W1[3:], ((0, 5), (0, 0)))

    idx1 = lax.cond(fits,
                    lambda: _knn_via_slots(posp128, 8),
                    lambda: _knn_full(posp, posp.T, br, bc))
    G1 = _sc_gather(posp128, idx1.T.reshape(-1))

    h2, st2 = _conv1_mid(G1, posp, w1a, w1b, b1[None, :],
                         g1[None, :], bt1[None, :], W2, b2[None, :])
    x1, A2, V2 = _conv1_tail(
        h2, st2, g2[None, :], bt2[None, :], W3, b3[None, :],
        W4[:64] - W4[64:], W4[64:], b4[None, :])

    x1p128 = jnp.pad(x1, ((0, 0), (0, 64)))
    idx2 = lax.cond(fits,
                    lambda: _knn_via_slots(x1p128, 64),
                    lambda: _knn_full(x1, x1.T, br, bc))
    M = _sc_maxgather(V2, idx2.T.reshape(-1))

    return _tail(x1, A2, M, br, W5[:64], W5[64:], b5[None, :],
                 W6, b6[None, :], W7, b7[None, :], W8, b8[None, :])


# speedup vs baseline: 5.1960x; 1.0007x over previous
import functools

import jax
import jax.numpy as jnp
from jax import lax
from jax.experimental import pallas as pl
from jax.experimental.pallas import tpu as pltpu
from jax.experimental.pallas import tpu_sc as plsc

N = 8192
K = 20
NUM_GRAPHS = 8
F32 = jnp.float32
MASKV = 1e37
NEDGE = N * K


def _dot(a, b):
    return lax.dot(a, b, precision=lax.Precision.DEFAULT,
                   preferred_element_type=F32)


def _extract_topk(dist_s, idx_ref, BR, W, base):
    colids = lax.broadcasted_iota(jnp.int32, (BR, W), 1)
    cols = []
    for _ in range(K):
        dcur = dist_s[...]
        m = jnp.min(dcur, axis=1, keepdims=True)
        cand = jnp.where(dcur == m, colids, jnp.int32(N))
        j = jnp.min(cand, axis=1, keepdims=True)
        cols.append(jnp.minimum(j + base, N - 1))
        dist_s[...] = jnp.where(colids == j, jnp.inf, dcur)
    idx_ref[...] = jnp.concatenate(cols, axis=1)


def _knn_full_body(x_ref, xt_ref, br_ref, bc_ref, idx_ref, dist_s, *, BR):
    xb = x_ref[...]
    sqr = jnp.sum(xb * xb, axis=1, keepdims=True)
    xt = xt_ref[...]
    sqc = jnp.sum(xt * xt, axis=0, keepdims=True)
    d = sqr + sqc - 2.0 * _dot(xb, xt)
    mask = br_ref[...] != bc_ref[...]
    dist_s[...] = jnp.where(mask, MASKV, d)
    _extract_topk(dist_s, idx_ref, BR, N, 0)


def _knn_full(x, xt, br, bc):
    BR = 64
    f = x.shape[1]
    return pl.pallas_call(
        functools.partial(_knn_full_body, BR=BR),
        grid=(N // BR,),
        in_specs=[
            pl.BlockSpec((BR, f), lambda i: (i, 0)),
            pl.BlockSpec((f, N), lambda i: (0, 0)),
            pl.BlockSpec((BR, 1), lambda i: (i, 0)),
            pl.BlockSpec((1, N), lambda i: (0, 0)),
        ],
        out_specs=pl.BlockSpec((BR, K), lambda i: (i, 0)),
        out_shape=jax.ShapeDtypeStruct((N, K), jnp.int32),
        scratch_shapes=[pltpu.VMEM((BR, N), F32)],
    )(x, xt, br, bc)


SLOT = 1280
NP = NUM_GRAPHS * SLOT


def _knn_pad_body(x_ref, xtg_ref, br_ref, bcg_ref, st_ref, idx_ref, dist_s,
                  *, BR):
    xb = x_ref[...]
    sqr = jnp.sum(xb * xb, axis=1, keepdims=True)
    xt = xtg_ref[0]
    sqc = jnp.sum(xt * xt, axis=0, keepdims=True)
    d = sqr + sqc - 2.0 * _dot(xb, xt)
    mask = br_ref[...] != bcg_ref[0]
    dist_s[...] = jnp.where(mask, MASKV, d)
    _extract_topk(dist_s, idx_ref, BR, SLOT, st_ref[0, 0, 0])


def _knn_pad(xp, xtg, brp, bcg, starts):
    BR = 256
    bpg = SLOT // BR
    f = xp.shape[1]
    return pl.pallas_call(
        functools.partial(_knn_pad_body, BR=BR),
        grid=(NP // BR,),
        in_specs=[
            pl.BlockSpec((BR, f), lambda i: (i, 0)),
            pl.BlockSpec((1, f, SLOT), lambda i: (i // bpg, 0, 0)),
            pl.BlockSpec((BR, 1), lambda i: (i, 0)),
            pl.BlockSpec((1, 1, SLOT), lambda i: (i // bpg, 0, 0)),
            pl.BlockSpec((1, 1, 1), lambda i: (i // bpg, 0, 0)),
        ],
        out_specs=pl.BlockSpec((BR, K), lambda i: (i, 0)),
        out_shape=jax.ShapeDtypeStruct((NP, K), jnp.int32),
        scratch_shapes=[pltpu.VMEM((BR, SLOT), F32)],
    )(xp, xtg, brp, bcg, starts)


def _sc_gather(V1, idxf):
    mesh = plsc.VectorSubcoreMesh(core_axis_name="c", subcore_axis_name="s")
    E = idxf.shape[0]
    chunk = E // 32
    CH = 128 if chunk % 128 == 0 else 64
    dt = V1.dtype

    @functools.partial(
        pl.kernel,
        mesh=mesh,
        out_type=jax.ShapeDtypeStruct((E, 128), dt),
        scratch_types=[
            pltpu.VMEM((CH,), jnp.int32),
            pltpu.VMEM((CH, 128), dt),
            pltpu.SemaphoreType.DMA,
        ],
    )
    def k_fn(v1_hbm, idx_hbm, g1_hbm, idx_v, rows_v, sem):
        wid = lax.axis_index("s") * 2 + lax.axis_index("c")
        i0 = wid * chunk

        def body(t, carry):
            eoff = i0 + t * CH
            pltpu.sync_copy(idx_hbm.at[pl.ds(eoff, CH)], idx_v)
            pltpu.async_copy(v1_hbm.at[idx_v], rows_v, sem).wait()
            pltpu.sync_copy(rows_v, g1_hbm.at[pl.ds(eoff, CH)])
            return carry

        lax.fori_loop(0, chunk // CH, body, 0)

    return k_fn(V1, idxf)


def _sc_maxgather(V2, idxf):
    mesh = plsc.VectorSubcoreMesh(core_axis_name="c", subcore_axis_name="s")
    CH = 128
    npts = N // 32

    @functools.partial(
        pl.kernel,
        mesh=mesh,
        out_type=jax.ShapeDtypeStruct((N, 128), F32),
        scratch_types=[
            pltpu.VMEM((CH,), jnp.int32),
            pltpu.VMEM((CH, 128), F32),
            pltpu.VMEM((CH, 128), F32),
            pltpu.SemaphoreType.DMA,
        ],
    )
    def k_fn(v2_hbm, idx_hbm, m_hbm, idx_v, rows_v, acc_v, sem):
        wid = lax.axis_index("s") * 2 + lax.axis_index("c")
        i0 = wid * npts

        def half(h, carry):
            base = i0 + h * CH

            def plane(k, c2):
                pltpu.sync_copy(idx_hbm.at[pl.ds(k * N + base, CH)], idx_v)
                @pl.when(k == 0)
                def _():
                    pltpu.async_copy(v2_hbm.at[idx_v], acc_v, sem).wait()

                @pl.when(k > 0)
                def _():
                    pltpu.async_copy(v2_hbm.at[idx_v], rows_v, sem).wait()

                    def row(r, c3):
                        for c in range(8):
                            sl = pl.ds(c * 16, 16)
                            acc_v[r, sl] = jnp.maximum(
                                acc_v[r, sl], rows_v[r, sl])
                        return c3

                    lax.fori_loop(0, CH, row, 0)
                return c2

            lax.fori_loop(0, K, plane, 0)
            pltpu.sync_copy(acc_v, m_hbm.at[pl.ds(base, CH)])
            return carry

        lax.fori_loop(0, npts // CH, half, 0)

    return k_fn(V2, idxf)


def _b_body(g1_ref, pos_ref, w1a_ref, w1b_ref, b1_ref, g_ref, bt_ref,
            w2_ref, b2_ref, h2_ref, st2_ref, st1_s):
    p = pl.program_id(0)
    k = pl.program_id(1)
    i = pl.program_id(2)
    first = (k == 0) & (i == 0)
    xi = pos_ref[...]
    h1 = (_dot(xi, w1a_ref[...])
          + _dot(g1_ref[:, :8] - xi, w1b_ref[...]) + b1_ref[...])

    @pl.when((p == 0) & first)
    def _():
        st1_s[...] = jnp.zeros_like(st1_s)

    @pl.when(p == 0)
    def _():
        st1_s[0:1, :] += jnp.sum(h1, axis=0, keepdims=True)
        st1_s[1:2, :] += jnp.sum(h1 * h1, axis=0, keepdims=True)

    @pl.when((p == 1) & first)
    def _():
        st2_ref[...] = jnp.zeros_like(st2_ref)

    @pl.when(p == 1)
    def _():
        mtot = jnp.float32(NEDGE)
        mean = st1_s[0:1, :] / mtot
        var = st1_s[1:2, :] / mtot - mean * mean
        al = g_ref[...] * lax.rsqrt(var + 1e-5)
        be = bt_ref[...] - al * mean
        y1 = jnp.maximum(al * h1 + be, 0.0)
        h2 = _dot(y1, w2_ref[...]) + b2_ref[...]
        h2_ref[...] = h2
        st2_ref[0:1, :] += jnp.sum(h2, axis=0, keepdims=True)
        st2_ref[1:2, :] += jnp.sum(h2 * h2, axis=0, keepdims=True)


def _conv1_mid(G1, posp, w1a, w1b, b1r, g1r, bt1r, W2, b2r):
    BR = 512
    nb = N // BR
    return pl.pallas_call(
        _b_body,
        grid=(2, K, nb),
        in_specs=[
            pl.BlockSpec((BR, 128), lambda p, k, i: (k * nb + i, 0)),
            pl.BlockSpec((BR, 8), lambda p, k, i: (i, 0)),
            pl.BlockSpec((8, 64), lambda p, k, i: (0, 0)),
            pl.BlockSpec((8, 64), lambda p, k, i: (0, 0)),
            pl.BlockSpec((1, 64), lambda p, k, i: (0, 0)),
            pl.BlockSpec((1, 64), lambda p, k, i: (0, 0)),
            pl.BlockSpec((1, 64), lambda p, k, i: (0, 0)),
            pl.BlockSpec((64, 64), lambda p, k, i: (0, 0)),
            pl.BlockSpec((1, 64), lambda p, k, i: (0, 0)),
        ],
        out_specs=[
            pl.BlockSpec((BR, 64), lambda p, k, i: (k * nb + i, 0)),
            pl.BlockSpec((2, 64), lambda p, k, i: (0, 0)),
        ],
        out_shape=[
            jax.ShapeDtypeStruct((NEDGE, 64), F32),
            jax.ShapeDtypeStruct((2, 64), F32),
        ],
        scratch_shapes=[pltpu.VMEM((2, 64), F32)],
    )(G1, posp, w1a, w1b, b1r, g1r, bt1r, W2, b2r)


def _c_body(h2_ref, st2_ref, g_ref, bt_ref, w3_ref, b3_ref,
            wd4_ref, w4b_ref, b4_ref, x1_ref, a2_ref, v2_ref, acc_s):
    k = pl.program_id(1)
    mtot = jnp.float32(NEDGE)
    mean = st2_ref[0:1, :] / mtot
    var = st2_ref[1:2, :] / mtot - mean * mean
    al = g_ref[...] * lax.rsqrt(var + 1e-5)
    be = bt_ref[...] - al * mean
    y2 = jnp.maximum(al * h2_ref[...] + be, 0.0)
    h3 = _dot(y2, w3_ref[...]) + b3_ref[...]

    @pl.when(k == 0)
    def _():
        acc_s[...] = h3

    @pl.when(k > 0)
    def _():
        acc_s[...] = jnp.maximum(acc_s[...], h3)

    @pl.when(k == K - 1)
    def _():
        x1b = acc_s[...]
        x1_ref[...] = x1b
        a2_ref[...] = _dot(x1b, wd4_ref[...]) + b4_ref[...]
        v2_ref[...] = _dot(x1b, w4b_ref[...])


def _conv1_tail(h2, st2, g2r, bt2r, W3, b3r, wd4, w4b, b4r):
    BR = 512
    nb = N // BR
    return pl.pallas_call(
        _c_body,
        grid=(nb, K),
        in_specs=[
            pl.BlockSpec((BR, 64), lambda i, k: (k * nb + i, 0)),
            pl.BlockSpec((2, 64), lambda i, k: (0, 0)),
            pl.BlockSpec((1, 64), lambda i, k: (0, 0)),
            pl.BlockSpec((1, 64), lambda i, k: (0, 0)),
            pl.BlockSpec((64, 64), lambda i, k: (0, 0)),
            pl.BlockSpec((1, 64), lambda i, k: (0, 0)),
            pl.BlockSpec((64, 128), lambda i, k: (0, 0)),
            pl.BlockSpec((64, 128), lambda i, k: (0, 0)),
            pl.BlockSpec((1, 128), lambda i, k: (0, 0)),
        ],
        out_specs=[
            pl.BlockSpec((BR, 64), lambda i, k: (i, 0)),
            pl.BlockSpec((BR, 128), lambda i, k: (i, 0)),
            pl.BlockSpec((BR, 128), lambda i, k: (i, 0)),
        ],
        out_shape=[
            jax.ShapeDtypeStruct((N, 64), F32),
            jax.ShapeDtypeStruct((N, 128), F32),
            jax.ShapeDtypeStruct((N, 128), F32),
        ],
        scratch_shapes=[pltpu.VMEM((BR, 64), F32)],
    )(h2, st2, g2r, bt2r, W3, b3r, wd4, w4b, b4r)


def _e_body(x1_ref, a2_ref, m_ref, b_ref, w5a_ref, w5b_ref, b5_ref,
            w6_ref, b6_ref, w7_ref, b7_ref, w8_ref, b8_ref,
            out_ref, pool_s, *, BR, NB):
    i = pl.program_id(0)

    @pl.when(i == 0)
    def _():
        pool_s[...] = jnp.full_like(pool_s, -jnp.inf)

    x2 = a2_ref[...] + m_ref[...]
    o = _dot(x1_ref[...], w5a_ref[...]) + _dot(x2, w5b_ref[...]) + b5_ref[...]
    bcol = b_ref[...]
    bmin = bcol[0, 0]
    bmax = bcol[BR - 1, 0]
    for s in range(NUM_GRAPHS):
        @pl.when((bmin <= s) & (s <= bmax))
        def _():
            seg = jnp.where(bcol == s, o, -jnp.inf)
            pool_s[s:s + 1, :] = jnp.maximum(
                pool_s[s:s + 1, :],
                jnp.max(seg, axis=0, keepdims=True))

    @pl.when(i == NB - 1)
    def _():
        h = jnp.maximum(_dot(pool_s[...], w6_ref[...]) + b6_ref[...], 0.0)
        h = jnp.maximum(_dot(h, w7_ref[...]) + b7_ref[...], 0.0)
        out_ref[...] = _dot(h, w8_ref[...]) + b8_ref[...]


def _tail(x1, A2, M, bcol, w5a, w5b, b5r, W6, b6r, W7, b7r, W8, b8r):
    BR = 512
    NB = N // BR
    return pl.pallas_call(
        functools.partial(_e_body, BR=BR, NB=NB),
        grid=(NB,),
        in_specs=[
            pl.BlockSpec((BR, 64), lambda i: (i, 0)),
            pl.BlockSpec((BR, 128), lambda i: (i, 0)),
            pl.BlockSpec((BR, 128), lambda i: (i, 0)),
            pl.BlockSpec((BR, 1), lambda i: (i, 0)),
            pl.BlockSpec((64, 1024), lambda i: (0, 0)),
            pl.BlockSpec((128, 1024), lambda i: (0, 0)),
            pl.BlockSpec((1, 1024), lambda i: (0, 0)),
            pl.BlockSpec((1024, 512), lambda i: (0, 0)),
            pl.BlockSpec((1, 512), lambda i: (0, 0)),
            pl.BlockSpec((512, 256), lambda i: (0, 0)),
            pl.BlockSpec((1, 256), lambda i: (0, 0)),
            pl.BlockSpec((256, 40), lambda i: (0, 0)),
            pl.BlockSpec((1, 40), lambda i: (0, 0)),
        ],
        out_specs=pl.BlockSpec((NUM_GRAPHS, 40), lambda i: (0, 0)),
        out_shape=jax.ShapeDtypeStruct((NUM_GRAPHS, 40), F32),
        scratch_shapes=[pltpu.VMEM((NUM_GRAPHS, 1024), F32)],
    )(x1, A2, M, bcol, w5a, w5b, b5r, W6, b6r, W7, b7r, W8, b8r)


def kernel(pos, batch, W1, b1, g1, bt1, W2, b2, g2, bt2, W3, b3,
           W4, b4, W5, b5, W6, b6, W7, b7, W8, b8):
    batch = batch.astype(jnp.int32)
    br = batch[:, None]
    bc = batch[None, :]

    gids = jnp.arange(NUM_GRAPHS, dtype=jnp.int32)
    starts = jnp.searchsorted(batch, gids).astype(jnp.int32)
    ends = jnp.searchsorted(batch, gids, side="right").astype(jnp.int32)
    counts = ends - starts
    fits = jnp.max(counts) <= SLOT
    slot = jnp.arange(NP, dtype=jnp.int32)
    sg = slot // SLOT
    soff = slot % SLOT
    p = jnp.minimum(starts[sg] + soff, N - 1)
    batch_pad = jnp.where(soff < counts[sg], sg, -1).astype(jnp.int32)
    brp = batch_pad[:, None]
    bcg = batch_pad.reshape(NUM_GRAPHS, 1, SLOT)
    starts2 = starts.reshape(NUM_GRAPHS, 1, 1)
    pinv = batch * SLOT + (jnp.arange(N, dtype=jnp.int32) - starts[batch])

    def _knn_via_slots(x128, f):
        xp = _sc_gather(x128, p)[:, :f]
        xtg = xp.T.reshape(f, NUM_GRAPHS, SLOT).transpose(1, 0, 2)
        idxp = _knn_pad(xp, xtg, brp, bcg, starts2)
        idxp128 = jnp.pad(idxp, ((0, 0), (0, 128 - K)))
        return _sc_gather(idxp128, pinv)[:, :K]

    posp = jnp.pad(pos, ((0, 0), (0, 5)))
    posp128 = jnp.pad(pos, ((0, 0), (0, 125)))
    w1a = jnp.pad(W1[:3], ((0, 5), (0, 0)))
    w1b = jnp.pad(W1[3:], ((0, 5), (0, 0)))

    idx1 = lax.cond(fits,
                    lambda: _knn_via_slots(posp128, 8),
                    lambda: _knn_full(posp, posp.T, br, bc))
    G1 = _sc_gather(posp128, idx1.T.reshape(-1))

    h2, st2 = _conv1_mid(G1, posp, w1a, w1b, b1[None, :],
                         g1[None, :], bt1[None, :], W2, b2[None, :])
    x1, A2, V2 = _conv1_tail(
        h2, st2, g2[None, :], bt2[None, :], W3, b3[None, :],
        W4[:64] - W4[64:], W4[64:], b4[None, :])

    x1p128 = jnp.pad(x1, ((0, 0), (0, 64)))
    idx2 = lax.cond(fits,
                    lambda: _knn_via_slots(x1p128, 64),
                    lambda: _knn_full(x1, x1.T, br, bc))
    M = _sc_maxgather(V2, idx2.T.reshape(-1))

    return _tail(x1, A2, M, br, W5[:64], W5[64:], b5[None, :],
                 W6, b6[None, :], W7, b7[None, :], W8, b8[None, :])


# speedup vs baseline: 5.4494x; 1.0488x over previous
import functools

import jax
import jax.numpy as jnp
from jax import lax
from jax.experimental import pallas as pl
from jax.experimental.pallas import tpu as pltpu
from jax.experimental.pallas import tpu_sc as plsc

N = 8192
K = 20
NUM_GRAPHS = 8
F32 = jnp.float32
MASKV = 1e37
NEDGE = N * K


def _dot(a, b):
    return lax.dot(a, b, precision=lax.Precision.DEFAULT,
                   preferred_element_type=F32)


def _extract_topk(dist_s, idx_ref, BR, W, base):
    colids = lax.broadcasted_iota(jnp.int32, (BR, W), 1)
    cols = []
    for _ in range(K):
        dcur = dist_s[...]
        m = jnp.min(dcur, axis=1, keepdims=True)
        cand = jnp.where(dcur == m, colids, jnp.int32(N))
        j = jnp.min(cand, axis=1, keepdims=True)
        cols.append(jnp.minimum(j + base, N - 1))
        dist_s[...] = jnp.where(colids == j, jnp.inf, dcur)
    idx_ref[...] = jnp.concatenate(cols, axis=1)


def _knn_full_body(x_ref, xt_ref, br_ref, bc_ref, idx_ref, dist_s, *, BR):
    xb = x_ref[...]
    sqr = jnp.sum(xb * xb, axis=1, keepdims=True)
    xt = xt_ref[...]
    sqc = jnp.sum(xt * xt, axis=0, keepdims=True)
    d = sqr + sqc - 2.0 * _dot(xb, xt)
    mask = br_ref[...] != bc_ref[...]
    dist_s[...] = jnp.where(mask, MASKV, d)
    _extract_topk(dist_s, idx_ref, BR, N, 0)


def _knn_full(x, xt, br, bc):
    BR = 64
    f = x.shape[1]
    return pl.pallas_call(
        functools.partial(_knn_full_body, BR=BR),
        grid=(N // BR,),
        in_specs=[
            pl.BlockSpec((BR, f), lambda i: (i, 0)),
            pl.BlockSpec((f, N), lambda i: (0, 0)),
            pl.BlockSpec((BR, 1), lambda i: (i, 0)),
            pl.BlockSpec((1, N), lambda i: (0, 0)),
        ],
        out_specs=pl.BlockSpec((BR, K), lambda i: (i, 0)),
        out_shape=jax.ShapeDtypeStruct((N, K), jnp.int32),
        scratch_shapes=[pltpu.VMEM((BR, N), F32)],
    )(x, xt, br, bc)


SLOT = 1280
NP = NUM_GRAPHS * SLOT


def _knn_pad_body(x_ref, xtg_ref, br_ref, bcg_ref, st_ref, idx_ref, dist_s,
                  *, BR):
    xb = x_ref[...]
    sqr = jnp.sum(xb * xb, axis=1, keepdims=True)
    xt = xtg_ref[0]
    sqc = jnp.sum(xt * xt, axis=0, keepdims=True)
    d = sqr + sqc - 2.0 * _dot(xb, xt)
    mask = br_ref[...] != bcg_ref[0]
    dist_s[...] = jnp.where(mask, MASKV, d)
    _extract_topk(dist_s, idx_ref, BR, SLOT, st_ref[0, 0, 0])


def _knn_pad(xp, xtg, brp, bcg, starts):
    BR = 640
    bpg = SLOT // BR
    f = xp.shape[1]
    return pl.pallas_call(
        functools.partial(_knn_pad_body, BR=BR),
        grid=(NP // BR,),
        in_specs=[
            pl.BlockSpec((BR, f), lambda i: (i, 0)),
            pl.BlockSpec((1, f, SLOT), lambda i: (i // bpg, 0, 0)),
            pl.BlockSpec((BR, 1), lambda i: (i, 0)),
            pl.BlockSpec((1, 1, SLOT), lambda i: (i // bpg, 0, 0)),
            pl.BlockSpec((1, 1, 1), lambda i: (i // bpg, 0, 0)),
        ],
        out_specs=pl.BlockSpec((BR, K), lambda i: (i, 0)),
        out_shape=jax.ShapeDtypeStruct((NP, K), jnp.int32),
        scratch_shapes=[pltpu.VMEM((BR, SLOT), F32)],
    )(xp, xtg, brp, bcg, starts)


def _sc_gather(V1, idxf):
    mesh = plsc.VectorSubcoreMesh(core_axis_name="c", subcore_axis_name="s")
    E = idxf.shape[0]
    chunk = E // 32
    CH = 128 if chunk % 128 == 0 else 64
    dt = V1.dtype

    @functools.partial(
        pl.kernel,
        mesh=mesh,
        out_type=jax.ShapeDtypeStruct((E, 128), dt),
        scratch_types=[
            pltpu.VMEM((CH,), jnp.int32),
            pltpu.VMEM((CH, 128), dt),
            pltpu.SemaphoreType.DMA,
        ],
    )
    def k_fn(v1_hbm, idx_hbm, g1_hbm, idx_v, rows_v, sem):
        wid = lax.axis_index("s") * 2 + lax.axis_index("c")
        i0 = wid * chunk

        def body(t, carry):
            eoff = i0 + t * CH
            pltpu.sync_copy(idx_hbm.at[pl.ds(eoff, CH)], idx_v)
            pltpu.async_copy(v1_hbm.at[idx_v], rows_v, sem).wait()
            pltpu.sync_copy(rows_v, g1_hbm.at[pl.ds(eoff, CH)])
            return carry

        lax.fori_loop(0, chunk // CH, body, 0)

    return k_fn(V1, idxf)


def _sc_maxgather(V2, idxf):
    mesh = plsc.VectorSubcoreMesh(core_axis_name="c", subcore_axis_name="s")
    CH = 128
    npts = N // 32

    @functools.partial(
        pl.kernel,
        mesh=mesh,
        out_type=jax.ShapeDtypeStruct((N, 128), F32),
        scratch_types=[
            pltpu.VMEM((CH,), jnp.int32),
            pltpu.VMEM((CH, 128), F32),
            pltpu.VMEM((CH, 128), F32),
            pltpu.SemaphoreType.DMA,
        ],
    )
    def k_fn(v2_hbm, idx_hbm, m_hbm, idx_v, rows_v, acc_v, sem):
        wid = lax.axis_index("s") * 2 + lax.axis_index("c")
        i0 = wid * npts

        def half(h, carry):
            base = i0 + h * CH

            def plane(k, c2):
                pltpu.sync_copy(idx_hbm.at[pl.ds(k * N + base, CH)], idx_v)
                @pl.when(k == 0)
                def _():
                    pltpu.async_copy(v2_hbm.at[idx_v], acc_v, sem).wait()

                @pl.when(k > 0)
                def _():
                    pltpu.async_copy(v2_hbm.at[idx_v], rows_v, sem).wait()

                    def row(r, c3):
                        for c in range(8):
                            sl = pl.ds(c * 16, 16)
                            acc_v[r, sl] = jnp.maximum(
                                acc_v[r, sl], rows_v[r, sl])
                        return c3

                    lax.fori_loop(0, CH, row, 0)
                return c2

            lax.fori_loop(0, K, plane, 0)
            pltpu.sync_copy(acc_v, m_hbm.at[pl.ds(base, CH)])
            return carry

        lax.fori_loop(0, npts // CH, half, 0)

    return k_fn(V2, idxf)


def _b_body(g1_ref, pos_ref, w1a_ref, w1b_ref, b1_ref, g_ref, bt_ref,
            w2_ref, b2_ref, h2_ref, st2_ref, st1_s):
    p = pl.program_id(0)
    k = pl.program_id(1)
    i = pl.program_id(2)
    first = (k == 0) & (i == 0)
    xi = pos_ref[...]
    h1 = (_dot(xi, w1a_ref[...])
          + _dot(g1_ref[:, :8] - xi, w1b_ref[...]) + b1_ref[...])

    @pl.when((p == 0) & first)
    def _():
        st1_s[...] = jnp.zeros_like(st1_s)

    @pl.when(p == 0)
    def _():
        st1_s[0:1, :] += jnp.sum(h1, axis=0, keepdims=True)
        st1_s[1:2, :] += jnp.sum(h1 * h1, axis=0, keepdims=True)

    @pl.when((p == 1) & first)
    def _():
        st2_ref[...] = jnp.zeros_like(st2_ref)

    @pl.when(p == 1)
    def _():
        mtot = jnp.float32(NEDGE)
        mean = st1_s[0:1, :] / mtot
        var = st1_s[1:2, :] / mtot - mean * mean
        al = g_ref[...] * lax.rsqrt(var + 1e-5)
        be = bt_ref[...] - al * mean
        y1 = jnp.maximum(al * h1 + be, 0.0)
        h2 = _dot(y1, w2_ref[...]) + b2_ref[...]
        h2_ref[...] = h2
        st2_ref[0:1, :] += jnp.sum(h2, axis=0, keepdims=True)
        st2_ref[1:2, :] += jnp.sum(h2 * h2, axis=0, keepdims=True)


def _conv1_mid(G1, posp, w1a, w1b, b1r, g1r, bt1r, W2, b2r):
    BR = 512
    nb = N // BR
    return pl.pallas_call(
        _b_body,
        grid=(2, K, nb),
        in_specs=[
            pl.BlockSpec((BR, 128), lambda p, k, i: (k * nb + i, 0)),
            pl.BlockSpec((BR, 8), lambda p, k, i: (i, 0)),
            pl.BlockSpec((8, 64), lambda p, k, i: (0, 0)),
            pl.BlockSpec((8, 64), lambda p, k, i: (0, 0)),
            pl.BlockSpec((1, 64), lambda p, k, i: (0, 0)),
            pl.BlockSpec((1, 64), lambda p, k, i: (0, 0)),
            pl.BlockSpec((1, 64), lambda p, k, i: (0, 0)),
            pl.BlockSpec((64, 64), lambda p, k, i: (0, 0)),
            pl.BlockSpec((1, 64), lambda p, k, i: (0, 0)),
        ],
        out_specs=[
            pl.BlockSpec((BR, 64), lambda p, k, i: (k * nb + i, 0)),
            pl.BlockSpec((2, 64), lambda p, k, i: (0, 0)),
        ],
        out_shape=[
            jax.ShapeDtypeStruct((NEDGE, 64), F32),
            jax.ShapeDtypeStruct((2, 64), F32),
        ],
        scratch_shapes=[pltpu.VMEM((2, 64), F32)],
    )(G1, posp, w1a, w1b, b1r, g1r, bt1r, W2, b2r)


def _c_body(h2_ref, st2_ref, g_ref, bt_ref, w3_ref, b3_ref,
            wd4_ref, w4b_ref, b4_ref, x1_ref, a2_ref, v2_ref, acc_s):
    k = pl.program_id(1)
    mtot = jnp.float32(NEDGE)
    mean = st2_ref[0:1, :] / mtot
    var = st2_ref[1:2, :] / mtot - mean * mean
    al = g_ref[...] * lax.rsqrt(var + 1e-5)
    be = bt_ref[...] - al * mean
    y2 = jnp.maximum(al * h2_ref[...] + be, 0.0)
    h3 = _dot(y2, w3_ref[...]) + b3_ref[...]

    @pl.when(k == 0)
    def _():
        acc_s[...] = h3

    @pl.when(k > 0)
    def _():
        acc_s[...] = jnp.maximum(acc_s[...], h3)

    @pl.when(k == K - 1)
    def _():
        x1b = acc_s[...]
        x1_ref[...] = x1b
        a2_ref[...] = _dot(x1b, wd4_ref[...]) + b4_ref[...]
        v2_ref[...] = _dot(x1b, w4b_ref[...])


def _conv1_tail(h2, st2, g2r, bt2r, W3, b3r, wd4, w4b, b4r):
    BR = 512
    nb = N // BR
    return pl.pallas_call(
        _c_body,
        grid=(nb, K),
        in_specs=[
            pl.BlockSpec((BR, 64), lambda i, k: (k * nb + i, 0)),
            pl.BlockSpec((2, 64), lambda i, k: (0, 0)),
            pl.BlockSpec((1, 64), lambda i, k: (0, 0)),
            pl.BlockSpec((1, 64), lambda i, k: (0, 0)),
            pl.BlockSpec((64, 64), lambda i, k: (0, 0)),
            pl.BlockSpec((1, 64), lambda i, k: (0, 0)),
            pl.BlockSpec((64, 128), lambda i, k: (0, 0)),
            pl.BlockSpec((64, 128), lambda i, k: (0, 0)),
            pl.BlockSpec((1, 128), lambda i, k: (0, 0)),
        ],
        out_specs=[
            pl.BlockSpec((BR, 64), lambda i, k: (i, 0)),
            pl.BlockSpec((BR, 128), lambda i, k: (i, 0)),
            pl.BlockSpec((BR, 128), lambda i, k: (i, 0)),
        ],
        out_shape=[
            jax.ShapeDtypeStruct((N, 64), F32),
            jax.ShapeDtypeStruct((N, 128), F32),
            jax.ShapeDtypeStruct((N, 128), F32),
        ],
        scratch_shapes=[pltpu.VMEM((BR, 64), F32)],
    )(h2, st2, g2r, bt2r, W3, b3r, wd4, w4b, b4r)


def _e_body(x1_ref, a2_ref, m_ref, b_ref, w5a_ref, w5b_ref, b5_ref,
            w6_ref, b6_ref, w7_ref, b7_ref, w8_ref, b8_ref,
            out_ref, pool_s, *, BR, NB):
    i = pl.program_id(0)

    @pl.when(i == 0)
    def _():
        pool_s[...] = jnp.full_like(pool_s, -jnp.inf)

    x2 = a2_ref[...] + m_ref[...]
    o = _dot(x1_ref[...], w5a_ref[...]) + _dot(x2, w5b_ref[...]) + b5_ref[...]
    bcol = b_ref[...]
    bmin = bcol[0, 0]
    bmax = bcol[BR - 1, 0]
    for s in range(NUM_GRAPHS):
        @pl.when((bmin <= s) & (s <= bmax))
        def _():
            seg = jnp.where(bcol == s, o, -jnp.inf)
            pool_s[s:s + 1, :] = jnp.maximum(
                pool_s[s:s + 1, :],
                jnp.max(seg, axis=0, keepdims=True))

    @pl.when(i == NB - 1)
    def _():
        h = jnp.maximum(_dot(pool_s[...], w6_ref[...]) + b6_ref[...], 0.0)
        h = jnp.maximum(_dot(h, w7_ref[...]) + b7_ref[...], 0.0)
        out_ref[...] = _dot(h, w8_ref[...]) + b8_ref[...]


def _tail(x1, A2, M, bcol, w5a, w5b, b5r, W6, b6r, W7, b7r, W8, b8r):
    BR = 512
    NB = N // BR
    return pl.pallas_call(
        functools.partial(_e_body, BR=BR, NB=NB),
        grid=(NB,),
        in_specs=[
            pl.BlockSpec((BR, 64), lambda i: (i, 0)),
            pl.BlockSpec((BR, 128), lambda i: (i, 0)),
            pl.BlockSpec((BR, 128), lambda i: (i, 0)),
            pl.BlockSpec((BR, 1), lambda i: (i, 0)),
            pl.BlockSpec((64, 1024), lambda i: (0, 0)),
            pl.BlockSpec((128, 1024), lambda i: (0, 0)),
            pl.BlockSpec((1, 1024), lambda i: (0, 0)),
            pl.BlockSpec((1024, 512), lambda i: (0, 0)),
            pl.BlockSpec((1, 512), lambda i: (0, 0)),
            pl.BlockSpec((512, 256), lambda i: (0, 0)),
            pl.BlockSpec((1, 256), lambda i: (0, 0)),
            pl.BlockSpec((256, 40), lambda i: (0, 0)),
            pl.BlockSpec((1, 40), lambda i: (0, 0)),
        ],
        out_specs=pl.BlockSpec((NUM_GRAPHS, 40), lambda i: (0, 0)),
        out_shape=jax.ShapeDtypeStruct((NUM_GRAPHS, 40), F32),
        scratch_shapes=[pltpu.VMEM((NUM_GRAPHS, 1024), F32)],
    )(x1, A2, M, bcol, w5a, w5b, b5r, W6, b6r, W7, b7r, W8, b8r)


def kernel(pos, batch, W1, b1, g1, bt1, W2, b2, g2, bt2, W3, b3,
           W4, b4, W5, b5, W6, b6, W7, b7, W8, b8):
    batch = batch.astype(jnp.int32)
    br = batch[:, None]
    bc = batch[None, :]

    gids = jnp.arange(NUM_GRAPHS, dtype=jnp.int32)
    starts = jnp.searchsorted(batch, gids).astype(jnp.int32)
    ends = jnp.searchsorted(batch, gids, side="right").astype(jnp.int32)
    counts = ends - starts
    fits = jnp.max(counts) <= SLOT
    slot = jnp.arange(NP, dtype=jnp.int32)
    sg = slot // SLOT
    soff = slot % SLOT
    p = jnp.minimum(starts[sg] + soff, N - 1)
    batch_pad = jnp.where(soff < counts[sg], sg, -1).astype(jnp.int32)
    brp = batch_pad[:, None]
    bcg = batch_pad.reshape(NUM_GRAPHS, 1, SLOT)
    starts2 = starts.reshape(NUM_GRAPHS, 1, 1)
    pinv = batch * SLOT + (jnp.arange(N, dtype=jnp.int32) - starts[batch])

    def _knn_via_slots(x128, f):
        xp = _sc_gather(x128, p)[:, :f]
        xtg = xp.T.reshape(f, NUM_GRAPHS, SLOT).transpose(1, 0, 2)
        idxp = _knn_pad(xp, xtg, brp, bcg, starts2)
        idxp128 = jnp.pad(idxp, ((0, 0), (0, 128 - K)))
        return _sc_gather(idxp128, pinv)[:, :K]

    posp = jnp.pad(pos, ((0, 0), (0, 5)))
    posp128 = jnp.pad(pos, ((0, 0), (0, 125)))
    w1a = jnp.pad(W1[:3], ((0, 5), (0, 0)))
    w1b = jnp.pad(W1[3:], ((0, 5), (0, 0)))

    idx1 = lax.cond(fits,
                    lambda: _knn_via_slots(posp128, 8),
                    lambda: _knn_full(posp, posp.T, br, bc))
    G1 = _sc_gather(posp128, idx1.T.reshape(-1))

    h2, st2 = _conv1_mid(G1, posp, w1a, w1b, b1[None, :],
                         g1[None, :], bt1[None, :], W2, b2[None, :])
    x1, A2, V2 = _conv1_tail(
        h2, st2, g2[None, :], bt2[None, :], W3, b3[None, :],
        W4[:64] - W4[64:], W4[64:], b4[None, :])

    x1p128 = jnp.pad(x1, ((0, 0), (0, 64)))
    idx2 = lax.cond(fits,
                    lambda: _knn_via_slots(x1p128, 64),
                    lambda: _knn_full(x1, x1.T, br, bc))
    M = _sc_maxgather(V2, idx2.T.reshape(-1))

    return _tail(x1, A2, M, br, W5[:64], W5[64:], b5[None, :],
                 W6, b6[None, :], W7, b7[None, :], W8, b8[None, :])
